# baseline (speedup 1.0000x reference)
import jax
import jax.numpy as jnp
from jax.experimental import pallas as pl
from jax.experimental.pallas import tpu as pltpu

N = 207
NP = 256
B = 64
U = 64
D0 = 2
D0P = 8
NM = 5
G = 32


def _kernel_body(x0_ref, adj_ref, wg0_ref, wc0_ref, wg1_ref, wc1_ref,
                 bcat_ref, h_ref,
                 s1_ref, s2_ref, w0_ref, b0_ref, w1_ref, b1_ref):
    bf16 = jnp.bfloat16

    @pl.when(pl.program_id(0) == 0)
    def _prep():
        valid = ((jax.lax.broadcasted_iota(jnp.int32, (NP, NP), 0) < N)
                 & (jax.lax.broadcasted_iota(jnp.int32, (NP, NP), 1) < N))
        adj = jnp.where(valid, adj_ref[...], 0.0)
        d1 = jnp.sum(adj, axis=1, keepdims=True)
        s1_ref[...] = jnp.where(d1 > 0.0, 1.0 / d1, 0.0) * adj
        adjt = jnp.transpose(adj)
        d2 = jnp.sum(adjt, axis=1, keepdims=True)
        s2_ref[...] = jnp.where(d2 > 0.0, 1.0 / d2, 0.0) * adjt

        def prep_w(wg_ref, wc_ref, din, dpad, wbd_ref):
            total_in = din + U
            wu = wg_ref[...].reshape(total_in, NM, 2 * U)[:din, :, U:]
            wc = wc_ref[...].reshape(total_in, NM, U)[:din]
            w = jnp.concatenate([0.5 * wu, wc], axis=2).astype(bf16)
            P = 128 // dpad
            wbd_ref[...] = jnp.zeros_like(wbd_ref)
            for m in range(NM):
                for e in range(P):
                    r = m * 128 + e * dpad
                    wbd_ref[r:r + din, e * 128:(e + 1) * 128] = w[:, m, :]

        prep_w(wg0_ref, wc0_ref, D0, D0P, w0_ref)
        prep_w(wg1_ref, wc1_ref, U, U, w1_ref)
        bcat = bcat_ref[...]
        bf0 = jnp.concatenate([0.5 * bcat[:, U:2 * U],
                               bcat[:, 2 * U:3 * U]], axis=1)
        bf1 = jnp.concatenate([0.5 * bcat[:, 4 * U:5 * U],
                               bcat[:, 5 * U:6 * U]], axis=1)
        b0_ref[...] = jnp.concatenate([bf0] * (128 // D0P), axis=1)
        b1_ref[...] = jnp.concatenate([bf1] * (128 // U), axis=1)

    s1 = s1_ref[...]
    s2 = s2_ref[...]

    def dcgru_layer(x0, w, b, din):
        P = 128 // din
        C = G // P
        x1a = jnp.dot(s1, x0, preferred_element_type=jnp.float32)
        x2a = 2.0 * jnp.dot(s1, x1a, preferred_element_type=jnp.float32) - x0
        x1b = jnp.dot(s2, x0, preferred_element_type=jnp.float32)
        x2b = 2.0 * jnp.dot(s2, x1b, preferred_element_type=jnp.float32) - x0
        cols = [jnp.concatenate([x[:, c * 128:(c + 1) * 128]
                                 for c in range(C)], axis=0).astype(bf16)
                for x in (x0, x1a, x2a, x1b, x2b)]
        z = jnp.concatenate(cols, axis=1)
        gg = jnp.dot(z, w, preferred_element_type=jnp.float32) + b
        th = jnp.tanh(gg)
        ths = jnp.concatenate([th[:, 64:], th[:, :64]], axis=1)
        hv = (0.5 - 0.5 * th) * ths
        return [hv[c * NP:(c + 1) * NP, e * 128:e * 128 + 64]
                for c in range(C) for e in range(P)]

    h0s = dcgru_layer(x0_ref[0], w0_ref[...], b0_ref[...], D0P)
    x1in = jnp.concatenate(h0s, axis=1)
    h1s = dcgru_layer(x1in, w1_ref[...], b1_ref[...], U)
    for g in range(G):
        h_ref[0, g] = h0s[g]
        h_ref[1, g] = h1s[g]


def kernel(inputs, adj_mx, nodevec1, nodevec2,
           W_gate_0, b_gate_0, W_cand_0, b_cand_0,
           W_gate_1, b_gate_1, W_cand_1, b_cand_1):
    f32 = jnp.float32
    bf16 = jnp.bfloat16
    x0 = jnp.pad(inputs.reshape(B, N, D0),
                 ((0, 0), (0, NP - N), (0, D0P - D0)))
    x0 = jnp.transpose(x0.reshape(B // G, G, NP, D0P),
                       (0, 2, 1, 3)).reshape(B // G, NP, G * D0P)

    full = lambda shape: pl.BlockSpec(shape, lambda c: (0,) * len(shape))
    h = pl.pallas_call(
        _kernel_body,
        grid=(B // G,),
        in_specs=[
            pl.BlockSpec((1, NP, G * D0P), lambda c: (c, 0, 0)),
            full((NP, NP)),
            full(((D0 + U) * NM, 2 * U)), full(((D0 + U) * NM, U)),
            full((2 * U * NM, 2 * U)), full((2 * U * NM, U)),
            full((1, 6 * U)),
        ],
        out_specs=pl.BlockSpec((2, G, NP, U), lambda c: (0, c, 0, 0)),
        out_shape=jax.ShapeDtypeStruct((2, B, N, U), f32),
        scratch_shapes=[
            pltpu.VMEM((NP, NP), f32), pltpu.VMEM((NP, NP), f32),
            pltpu.VMEM((NM * 128, (128 // D0P) * 2 * U), bf16),
            pltpu.VMEM((1, (128 // D0P) * 2 * U), f32),
            pltpu.VMEM((NM * 128, (128 // U) * 2 * U), bf16),
            pltpu.VMEM((1, (128 // U) * 2 * U), f32),
        ],
    )(x0, adj_mx, W_gate_0, W_cand_0, W_gate_1, W_cand_1,
      jnp.concatenate([b_gate_0, b_cand_0,
                       b_gate_1, b_cand_1]).reshape(1, 6 * U))

    hidden = h.reshape(2, B, N * U)
    return hidden[1], hidden


# speedup vs baseline: 1.0347x; 1.0347x over previous
import jax
import jax.numpy as jnp
from jax.experimental import pallas as pl
from jax.experimental.pallas import tpu as pltpu

N = 207
NP = 256
B = 64
U = 64
D0 = 2
D0P = 8
NM = 5
G = 32


def _kernel_body(x0_ref, adj_ref, wg0_ref, wc0_ref, bg0_ref, bc0_ref,
                 wg1_ref, wc1_ref, bg1_ref, bc1_ref, h_ref,
                 s1_ref, s2_ref, w0_ref, b0_ref, w1_ref, b1_ref):
    bf16 = jnp.bfloat16

    @pl.when(pl.program_id(0) == 0)
    def _prep():
        valid = ((jax.lax.broadcasted_iota(jnp.int32, (NP, NP), 0) < N)
                 & (jax.lax.broadcasted_iota(jnp.int32, (NP, NP), 1) < N))
        adj = jnp.where(valid, adj_ref[...], 0.0)
        d1 = jnp.sum(adj, axis=1, keepdims=True)
        s1_ref[...] = jnp.where(d1 > 0.0, 1.0 / d1, 0.0) * adj
        adjt = jnp.transpose(adj)
        d2 = jnp.sum(adjt, axis=1, keepdims=True)
        s2_ref[...] = jnp.where(d2 > 0.0, 1.0 / d2, 0.0) * adjt

        def prep_w(wg_ref, wc_ref, din, dpad, wbd_ref):
            total_in = din + U
            wu = wg_ref[...].reshape(total_in, NM, 2 * U)[:din, :, U:]
            wc = wc_ref[...].reshape(total_in, NM, U)[:din]
            w = jnp.concatenate([0.5 * wu, wc], axis=2).astype(bf16)
            P = 128 // dpad
            wbd_ref[...] = jnp.zeros_like(wbd_ref)
            for m in range(NM):
                for e in range(P):
                    r = m * 128 + e * dpad
                    wbd_ref[r:r + din, e * 128:(e + 1) * 128] = w[:, m, :]

        prep_w(wg0_ref, wc0_ref, D0, D0P, w0_ref)
        prep_w(wg1_ref, wc1_ref, U, U, w1_ref)
        bf0 = jnp.concatenate([0.5 * bg0_ref[:, U:], bc0_ref[...]], axis=1)
        bf1 = jnp.concatenate([0.5 * bg1_ref[:, U:], bc1_ref[...]], axis=1)
        b0_ref[...] = jnp.concatenate([bf0] * (128 // D0P), axis=1)
        b1_ref[...] = jnp.concatenate([bf1] * (128 // U), axis=1)

    s1 = s1_ref[...]
    s2 = s2_ref[...]

    def dcgru_layer(x0, w, b, din):
        P = 128 // din
        C = G // P
        x1a = jnp.dot(s1, x0, preferred_element_type=jnp.float32)
        x2a = 2.0 * jnp.dot(s1, x1a, preferred_element_type=jnp.float32) - x0
        x1b = jnp.dot(s2, x0, preferred_element_type=jnp.float32)
        x2b = 2.0 * jnp.dot(s2, x1b, preferred_element_type=jnp.float32) - x0
        cols = [jnp.concatenate([x[:, c * 128:(c + 1) * 128]
                                 for c in range(C)], axis=0).astype(bf16)
                for x in (x0, x1a, x2a, x1b, x2b)]
        z = jnp.concatenate(cols, axis=1)
        gg = jnp.dot(z, w, preferred_element_type=jnp.float32) + b
        th = jnp.tanh(gg)
        ths = jnp.concatenate([th[:, 64:], th[:, :64]], axis=1)
        hv = (0.5 - 0.5 * th) * ths
        return [hv[c * NP:(c + 1) * NP, e * 128:e * 128 + 64]
                for c in range(C) for e in range(P)]

    h0s = dcgru_layer(x0_ref[0], w0_ref[...], b0_ref[...], D0P)
    x1in = jnp.concatenate(h0s, axis=1)
    h1s = dcgru_layer(x1in, w1_ref[...], b1_ref[...], U)
    for g in range(G):
        h_ref[0, g] = h0s[g]
        h_ref[1, g] = h1s[g]


def kernel(inputs, adj_mx, nodevec1, nodevec2,
           W_gate_0, b_gate_0, W_cand_0, b_cand_0,
           W_gate_1, b_gate_1, W_cand_1, b_cand_1):
    f32 = jnp.float32
    bf16 = jnp.bfloat16
    x0 = jnp.pad(inputs.reshape(B, N, D0),
                 ((0, 0), (0, NP - N), (0, D0P - D0)))
    x0 = jnp.transpose(x0.reshape(B // G, G, NP, D0P),
                       (0, 2, 1, 3)).reshape(B // G, NP, G * D0P)

    full = lambda shape: pl.BlockSpec(shape, lambda c: (0,) * len(shape))
    h = pl.pallas_call(
        _kernel_body,
        grid=(B // G,),
        in_specs=[
            pl.BlockSpec((1, NP, G * D0P), lambda c: (c, 0, 0)),
            full((NP, NP)),
            full(((D0 + U) * NM, 2 * U)), full(((D0 + U) * NM, U)),
            full((1, 2 * U)), full((1, U)),
            full((2 * U * NM, 2 * U)), full((2 * U * NM, U)),
            full((1, 2 * U)), full((1, U)),
        ],
        out_specs=pl.BlockSpec((2, G, NP, U), lambda c: (0, c, 0, 0)),
        out_shape=jax.ShapeDtypeStruct((2, B, N, U), f32),
        scratch_shapes=[
            pltpu.VMEM((NP, NP), f32), pltpu.VMEM((NP, NP), f32),
            pltpu.VMEM((NM * 128, (128 // D0P) * 2 * U), bf16),
            pltpu.VMEM((1, (128 // D0P) * 2 * U), f32),
            pltpu.VMEM((NM * 128, (128 // U) * 2 * U), bf16),
            pltpu.VMEM((1, (128 // U) * 2 * U), f32),
        ],
    )(x0, adj_mx,
      W_gate_0, W_cand_0, b_gate_0.reshape(1, 2 * U), b_cand_0.reshape(1, U),
      W_gate_1, W_cand_1, b_gate_1.reshape(1, 2 * U), b_cand_1.reshape(1, U))

    hidden = h.reshape(2, B, N * U)
    return hidden[1], hidden
